# baseline (speedup 1.0000x reference)
import functools

import jax
import jax.numpy as jnp
from jax import lax
from jax.experimental import pallas as pl
from jax.experimental.pallas import tpu as pltpu
from jax.experimental.pallas import tpu_sc as plsc

B, L, EMB = 4096, 200, 128
H1, H2, OUT = 512, 256, 2
VOCAB = 100000
NC, NS = 2, 16
NW = NC * NS
BPW = B // NW
LCA, LCB = 104, 96
EMBW = EMB // 2
NVW = EMBW // 16


def _make_pool_kernel():
    mesh = plsc.VectorSubcoreMesh(core_axis_name="c", subcore_axis_name="s")

    @functools.partial(
        pl.kernel,
        mesh=mesh,
        out_type=jax.ShapeDtypeStruct((B, EMB), jnp.float32),
        compiler_params=pltpu.CompilerParams(use_tc_tiling_on_sc=False),
        scratch_types=[
            pltpu.VMEM((BPW * L,), jnp.int32),
            pltpu.VMEM((LCA, EMBW), jnp.int32),
            pltpu.VMEM((LCB, EMBW), jnp.int32),
            pltpu.VMEM((LCA, EMBW), jnp.int32),
            pltpu.VMEM((LCB, EMBW), jnp.int32),
            pltpu.VMEM((BPW, EMB), jnp.float32),
            pltpu.SemaphoreType.DMA,
            pltpu.SemaphoreType.DMA,
            pltpu.SemaphoreType.DMA,
            pltpu.SemaphoreType.DMA,
        ],
    )
    def pool(emb_hbm, xr_hbm, out_hbm, idx_v,
             bufa0, bufb0, bufa1, bufb1, out_v, sa0, sb0, sa1, sb1):
        wid = lax.axis_index("s") * NC + lax.axis_index("c")
        base = wid * BPW
        pltpu.sync_copy(xr_hbm.at[pl.ds(base * L, BPW * L)], idx_v)

        bufs = ((bufa0, bufb0, sa0, sb0), (bufa1, bufb1, sa1, sb1))
        himask = jnp.full((16,), -65536, jnp.int32)

        def start(i, s):
            bufa, bufb, sema, semb = bufs[s]
            pltpu.async_copy(
                emb_hbm.at[idx_v.at[pl.ds(i * L, LCA)]], bufa, sema)
            pltpu.async_copy(
                emb_hbm.at[idx_v.at[pl.ds(i * L + LCA, LCB)]], bufb, semb)

        def finish(i, s):
            bufa, bufb, sema, semb = bufs[s]
            pltpu.make_async_copy(
                emb_hbm.at[idx_v.at[pl.ds(i * L, LCA)]], bufa, sema).wait()
            pltpu.make_async_copy(
                emb_hbm.at[idx_v.at[pl.ds(i * L + LCA, LCB)]], bufb,
                semb).wait()

            def expand_add(buf, r, acc):
                acc = list(acc)
                for c in range(NVW):
                    w = buf[r, pl.ds(c * 16, 16)]
                    lo = lax.bitcast_convert_type(
                        jnp.left_shift(w, 16), jnp.float32)
                    hi = lax.bitcast_convert_type(
                        jnp.bitwise_and(w, himask), jnp.float32)
                    acc[c] = acc[c] + lo
                    acc[NVW + c] = acc[NVW + c] + hi
                return tuple(acc)

            def red2(r, acc):
                return expand_add(bufb, r, expand_add(bufa, r, acc))

            def red1(r, acc):
                return expand_add(bufa, r, acc)

            acc = tuple(jnp.zeros((16,), jnp.float32) for _ in range(2 * NVW))
            acc = lax.fori_loop(0, LCB, red2, acc)
            acc = lax.fori_loop(LCB, LCA, red1, acc)
            for c in range(NVW):
                out_v[i, pl.ds(16 * c, 16)] = acc[c]
                out_v[i, pl.ds(64 + 16 * c, 16)] = acc[NVW + c]

        start(0, 0)

        def pair(p, carry):
            i = 2 * p
            start(i + 1, 1)
            finish(i, 0)

            @pl.when(p < BPW // 2 - 1)
            def _():
                start(i + 2, 0)

            finish(i + 1, 1)
            return carry

        lax.fori_loop(0, BPW // 2, pair, 0)
        pltpu.sync_copy(out_v, out_hbm.at[pl.ds(base, BPW)])

    return pool


_pool = _make_pool_kernel()


def _pack_table(emb):
    R = 10000
    HV = VOCAB // 2

    def body(a_ref, b_ref, o_ref):
        rnd = jnp.uint32(0x8000)
        himask = jnp.uint32(0xFFFF0000)

        def pack(x):
            u = lax.bitcast_convert_type(x, jnp.uint32) + rnd
            lo = jnp.right_shift(u[:, :EMBW], 16)
            hi = jnp.bitwise_and(u[:, EMBW:], himask)
            return lax.bitcast_convert_type(jnp.bitwise_or(lo, hi),
                                            jnp.int32)

        o_ref[:, :EMBW] = pack(a_ref[...])
        o_ref[:, EMBW:] = pack(b_ref[...])

    return pl.pallas_call(
        body,
        grid=(HV // R,),
        in_specs=[
            pl.BlockSpec((R, EMB), lambda i: (i, 0)),
            pl.BlockSpec((R, EMB), lambda i: (i + HV // R, 0)),
        ],
        out_specs=pl.BlockSpec((R, EMB), lambda i: (i, 0)),
        out_shape=jax.ShapeDtypeStruct((HV, EMB), jnp.int32),
    )(emb, emb)


def _mlp(x, W1, b1, W2, b2, W3, b3):
    BT = 512

    def body(x_ref, w1_ref, b1_ref, w2_ref, b2_ref, w3_ref, b3_ref, o_ref):
        bf = jnp.bfloat16

        def dot(a, b):
            return lax.dot_general(a.astype(bf), b.astype(bf),
                                   (((1,), (1,)), ((), ())),
                                   preferred_element_type=jnp.float32)

        h = x_ref[...] * (1.0 / L)
        h = dot(h, w1_ref[...]) + b1_ref[...]
        h = jnp.maximum(h, 0.0)
        h = dot(h, w2_ref[...]) + b2_ref[...]
        h = jnp.maximum(h, 0.0)
        h = dot(h, w3_ref[...]) + b3_ref[...]
        o_ref[...] = h

    return pl.pallas_call(
        body,
        grid=(B // BT,),
        in_specs=[
            pl.BlockSpec((BT, EMB), lambda i: (i, 0)),
            pl.BlockSpec((H1, EMB), lambda i: (0, 0)),
            pl.BlockSpec((1, H1), lambda i: (0, 0)),
            pl.BlockSpec((H2, H1), lambda i: (0, 0)),
            pl.BlockSpec((1, H2), lambda i: (0, 0)),
            pl.BlockSpec((OUT, H2), lambda i: (0, 0)),
            pl.BlockSpec((1, OUT), lambda i: (0, 0)),
        ],
        out_specs=pl.BlockSpec((BT, OUT), lambda i: (i, 0)),
        out_shape=jax.ShapeDtypeStruct((B, OUT), jnp.float32),
    )(x, W1, b1, W2, b2, W3, b3)


def kernel(X, emb, W1, b1, W2, b2, W3, b3):
    Xi = X.astype(jnp.int32)
    xr = jnp.where(Xi < VOCAB // 2, 2 * Xi, 2 * Xi - (VOCAB - 1))
    xr = xr.reshape(B * L)
    emb_w = _pack_table(emb).reshape(VOCAB, EMBW)
    pooled = _pool(emb_w, xr)
    return _mlp(pooled, W1, b1.reshape(1, H1), W2, b2.reshape(1, H2),
                W3, b3.reshape(1, OUT))


# speedup vs baseline: 1.0080x; 1.0080x over previous
import functools

import jax
import jax.numpy as jnp
from jax import lax
from jax.experimental import pallas as pl
from jax.experimental.pallas import tpu as pltpu
from jax.experimental.pallas import tpu_sc as plsc

B, L, EMB = 4096, 200, 128
H1, H2, OUT = 512, 256, 2
VOCAB = 100000
NC, NS = 2, 16
NW = NC * NS
BPW = B // NW
LCA, LCB = 104, 96
EMBW = EMB // 2
NVW = EMBW // 16


def _make_pool_kernel():
    mesh = plsc.VectorSubcoreMesh(core_axis_name="c", subcore_axis_name="s")

    @functools.partial(
        pl.kernel,
        mesh=mesh,
        out_type=jax.ShapeDtypeStruct((B, EMB), jnp.float32),
        compiler_params=pltpu.CompilerParams(use_tc_tiling_on_sc=False),
        scratch_types=[
            pltpu.VMEM((BPW * L,), jnp.int32),
            pltpu.VMEM((LCA, EMBW), jnp.int32),
            pltpu.VMEM((LCB, EMBW), jnp.int32),
            pltpu.VMEM((LCA, EMBW), jnp.int32),
            pltpu.VMEM((LCB, EMBW), jnp.int32),
            pltpu.VMEM((BPW, EMB), jnp.float32),
            pltpu.SemaphoreType.DMA,
            pltpu.SemaphoreType.DMA,
            pltpu.SemaphoreType.DMA,
            pltpu.SemaphoreType.DMA,
        ],
    )
    def pool(emb_hbm, xr_hbm, out_hbm, idx_v,
             bufa0, bufb0, bufa1, bufb1, out_v, sa0, sb0, sa1, sb1):
        wid = lax.axis_index("s") * NC + lax.axis_index("c")
        base = wid * BPW
        pltpu.sync_copy(xr_hbm.at[pl.ds(base * L, BPW * L)], idx_v)

        bufs = ((bufa0, bufb0, sa0, sb0), (bufa1, bufb1, sa1, sb1))
        himask = jnp.full((16,), -65536, jnp.int32)

        def start(i, s):
            bufa, bufb, sema, semb = bufs[s]
            pltpu.async_copy(
                emb_hbm.at[idx_v.at[pl.ds(i * L, LCA)]], bufa, sema)
            pltpu.async_copy(
                emb_hbm.at[idx_v.at[pl.ds(i * L + LCA, LCB)]], bufb, semb)

        def finish(i, s):
            bufa, bufb, sema, semb = bufs[s]
            pltpu.make_async_copy(
                emb_hbm.at[idx_v.at[pl.ds(i * L, LCA)]], bufa, sema).wait()
            pltpu.make_async_copy(
                emb_hbm.at[idx_v.at[pl.ds(i * L + LCA, LCB)]], bufb,
                semb).wait()

            def expand_add(buf, r, acc):
                acc = list(acc)
                for c in range(NVW):
                    w = buf[r, pl.ds(c * 16, 16)]
                    lo = lax.bitcast_convert_type(
                        jnp.left_shift(w, 16), jnp.float32)
                    hi = lax.bitcast_convert_type(
                        jnp.bitwise_and(w, himask), jnp.float32)
                    acc[c] = acc[c] + lo
                    acc[NVW + c] = acc[NVW + c] + hi
                return tuple(acc)

            def red2(r, acc):
                return expand_add(bufb, r, expand_add(bufa, r, acc))

            def red1(r, acc):
                return expand_add(bufa, r, acc)

            acc = tuple(jnp.zeros((16,), jnp.float32) for _ in range(2 * NVW))
            acc = lax.fori_loop(0, LCB, red2, acc)
            acc = lax.fori_loop(LCB, LCA, red1, acc)
            for c in range(NVW):
                out_v[i, pl.ds(16 * c, 16)] = acc[c]
                out_v[i, pl.ds(64 + 16 * c, 16)] = acc[NVW + c]

        start(0, 0)

        def pair(p, carry):
            i = 2 * p
            start(i + 1, 1)
            finish(i, 0)

            @pl.when(p < BPW // 2 - 1)
            def _():
                start(i + 2, 0)

            finish(i + 1, 1)
            return carry

        lax.fori_loop(0, BPW // 2, pair, 0)
        pltpu.sync_copy(out_v, out_hbm.at[pl.ds(base, BPW)])

    return pool


_pool = _make_pool_kernel()


def _pack_table(emb):
    R = 5000
    HV = VOCAB // 2

    def body(a_ref, b_ref, o_ref):
        rnd = jnp.uint32(0x8000)
        himask = jnp.uint32(0xFFFF0000)

        def pack(x):
            u = lax.bitcast_convert_type(x, jnp.uint32) + rnd
            lo = jnp.right_shift(u[:, :EMBW], 16)
            hi = jnp.bitwise_and(u[:, EMBW:], himask)
            return lax.bitcast_convert_type(jnp.bitwise_or(lo, hi),
                                            jnp.int32)

        o_ref[:, :EMBW] = pack(a_ref[...])
        o_ref[:, EMBW:] = pack(b_ref[...])

    return pl.pallas_call(
        body,
        grid=(HV // R,),
        in_specs=[
            pl.BlockSpec((R, EMB), lambda i: (i, 0)),
            pl.BlockSpec((R, EMB), lambda i: (i + HV // R, 0)),
        ],
        out_specs=pl.BlockSpec((R, EMB), lambda i: (i, 0)),
        out_shape=jax.ShapeDtypeStruct((HV, EMB), jnp.int32),
    )(emb, emb)


def _mlp(x, W1, b1, W2, b2, W3, b3):
    BT = 512

    def body(x_ref, w1_ref, b1_ref, w2_ref, b2_ref, w3_ref, b3_ref, o_ref):
        bf = jnp.bfloat16

        def dot(a, b):
            return lax.dot_general(a.astype(bf), b.astype(bf),
                                   (((1,), (1,)), ((), ())),
                                   preferred_element_type=jnp.float32)

        h = x_ref[...] * (1.0 / L)
        h = dot(h, w1_ref[...]) + b1_ref[...]
        h = jnp.maximum(h, 0.0)
        h = dot(h, w2_ref[...]) + b2_ref[...]
        h = jnp.maximum(h, 0.0)
        h = dot(h, w3_ref[...]) + b3_ref[...]
        o_ref[...] = h

    return pl.pallas_call(
        body,
        grid=(B // BT,),
        in_specs=[
            pl.BlockSpec((BT, EMB), lambda i: (i, 0)),
            pl.BlockSpec((H1, EMB), lambda i: (0, 0)),
            pl.BlockSpec((1, H1), lambda i: (0, 0)),
            pl.BlockSpec((H2, H1), lambda i: (0, 0)),
            pl.BlockSpec((1, H2), lambda i: (0, 0)),
            pl.BlockSpec((OUT, H2), lambda i: (0, 0)),
            pl.BlockSpec((1, OUT), lambda i: (0, 0)),
        ],
        out_specs=pl.BlockSpec((BT, OUT), lambda i: (i, 0)),
        out_shape=jax.ShapeDtypeStruct((B, OUT), jnp.float32),
    )(x, W1, b1, W2, b2, W3, b3)


def kernel(X, emb, W1, b1, W2, b2, W3, b3):
    Xi = X.astype(jnp.int32)
    xr = jnp.where(Xi < VOCAB // 2, 2 * Xi, 2 * Xi - (VOCAB - 1))
    xr = xr.reshape(B * L)
    emb_w = _pack_table(emb).reshape(VOCAB, EMBW)
    pooled = _pool(emb_w, xr)
    return _mlp(pooled, W1, b1.reshape(1, H1), W2, b2.reshape(1, H2),
                W3, b3.reshape(1, OUT))
